# baseline (speedup 1.0000x reference)
import jax
import jax.numpy as jnp
from jax import lax
from jax.experimental import pallas as pl
from jax.experimental.pallas import tpu as pltpu

N_FRAMES = 512
N_PIX = 128
N_CH = 3
BATCH = 64
NBUF = 4


def _sig(x):
    return 0.5 + 0.5 * jnp.tanh(0.5 * x)


def _copies(idx_ref, hbm, scr, sems, out_hbm, out_scr, sem_out, item, slot):
    f = idx_ref[item]
    mmu_h, mlv_h, h_h, ns_h, nh_h = hbm
    mmu_s, mlv_s, h_s, ns_s, nh_s = scr
    return [
        pltpu.make_async_copy(mmu_h.at[f], mmu_s.at[slot], sems[0].at[slot]),
        pltpu.make_async_copy(mlv_h.at[f], mlv_s.at[slot], sems[1].at[slot]),
        pltpu.make_async_copy(h_h.at[f], h_s.at[slot], sems[2].at[slot]),
        pltpu.make_async_copy(ns_h.at[item], ns_s.at[slot], sems[3].at[slot]),
        pltpu.make_async_copy(nh_h.at[item], nh_s.at[slot], sems[4].at[slot]),
    ]


def _rtvf_body(idx_ref, mmu_h, mlv_h, c_ref, h_h, b_ref, v_ref, ns_h, nh_h,
               out_h,
               mmu_s, mlv_s, h_s, ns_s, nh_s, out_s,
               sem0, sem1, sem2, sem3, sem4, sem_out):
    hbm = (mmu_h, mlv_h, h_h, ns_h, nh_h)
    scr = (mmu_s, mlv_s, h_s, ns_s, nh_s)
    sems = (sem0, sem1, sem2, sem3, sem4)

    def start_in(item, slot):
        for cp in _copies(idx_ref, hbm, scr, sems, out_h, out_s, sem_out,
                          item, slot):
            cp.start()

    def wait_in(item, slot):
        for cp in _copies(idx_ref, hbm, scr, sems, out_h, out_s, sem_out,
                          item, slot):
            cp.wait()

    for i in range(NBUF - 1):
        start_in(i, i)

    def step(i, _):
        slot = lax.rem(i, NBUF)
        nxt = i + NBUF - 1

        @pl.when(nxt < BATCH)
        def _():
            start_in(nxt, lax.rem(nxt, NBUF))

        wait_in(i, slot)

        f = idx_ref[i]
        c = c_ref[0, f]
        th = jnp.tanh(0.5 * mmu_s[slot])
        a = 0.5 - 0.5 * th
        one_m_a = 0.5 + 0.5 * th
        e = jnp.exp(mlv_s[slot])
        ns = ns_s[slot]
        nh = nh_s[slot]

        @pl.when(i >= NBUF)
        def _():
            pltpu.make_async_copy(out_s.at[slot], out_h.at[i - NBUF],
                                  sem_out.at[slot]).wait()

        for ch in range(N_CH):
            s = b_ref[ch] + c * v_ref[0, ch]
            hc = h_s[slot, ch]
            t = a * s + one_m_a * hc + e * (s * s * ns + hc * hc * nh)
            out_s[slot, ch] = _sig(t)

        pltpu.make_async_copy(out_s.at[slot], out_h.at[i],
                              sem_out.at[slot]).start()
        return 0

    lax.fori_loop(0, BATCH, step, 0)

    for i in range(BATCH - NBUF, BATCH):
        slot = i % NBUF
        pltpu.make_async_copy(out_s.at[slot], out_h.at[i],
                              sem_out.at[slot]).wait()


def kernel(index, img, B, V, C, Mmu, Mlv, H, noise_S, noise_H):
    del img
    idx = index.astype(jnp.int32)
    ht = jnp.transpose(H, (0, 3, 1, 2))
    vt = jnp.transpose(V, (0, 3, 1, 2))
    bt = jnp.transpose(B, (2, 0, 1))
    ct = jnp.transpose(C, (1, 0))
    ns = noise_S.reshape(BATCH, N_PIX, N_PIX)
    nh = noise_H.reshape(BATCH, N_PIX, N_PIX)

    any_spec = pl.BlockSpec(memory_space=pl.ANY)
    grid_spec = pltpu.PrefetchScalarGridSpec(
        num_scalar_prefetch=1,
        grid=(1,),
        in_specs=[
            any_spec,
            any_spec,
            pl.BlockSpec(memory_space=pltpu.SMEM),
            any_spec,
            pl.BlockSpec(memory_space=pltpu.VMEM),
            pl.BlockSpec(memory_space=pltpu.VMEM),
            any_spec,
            any_spec,
        ],
        out_specs=any_spec,
        scratch_shapes=[
            pltpu.VMEM((NBUF, N_PIX, N_PIX), jnp.float32),
            pltpu.VMEM((NBUF, N_PIX, N_PIX), jnp.float32),
            pltpu.VMEM((NBUF, N_CH, N_PIX, N_PIX), jnp.float32),
            pltpu.VMEM((NBUF, N_PIX, N_PIX), jnp.float32),
            pltpu.VMEM((NBUF, N_PIX, N_PIX), jnp.float32),
            pltpu.VMEM((NBUF, N_CH, N_PIX, N_PIX), jnp.float32),
            pltpu.SemaphoreType.DMA((NBUF,)),
            pltpu.SemaphoreType.DMA((NBUF,)),
            pltpu.SemaphoreType.DMA((NBUF,)),
            pltpu.SemaphoreType.DMA((NBUF,)),
            pltpu.SemaphoreType.DMA((NBUF,)),
            pltpu.SemaphoreType.DMA((NBUF,)),
        ],
    )

    out = pl.pallas_call(
        _rtvf_body,
        grid_spec=grid_spec,
        out_shape=jax.ShapeDtypeStruct((BATCH, N_CH, N_PIX, N_PIX),
                                       jnp.float32),
        compiler_params=pltpu.CompilerParams(
            dimension_semantics=("arbitrary",),
        ),
    )(idx, Mmu, Mlv, ct, ht, bt, vt, ns, nh)

    return jnp.transpose(out, (0, 2, 3, 1))


# speedup vs baseline: 1.4681x; 1.4681x over previous
import jax
import jax.numpy as jnp
from jax.experimental import pallas as pl
from jax.experimental.pallas import tpu as pltpu

N_FRAMES = 512
N_PIX = 128
N_CH = 3
BATCH = 64
PER_STEP = 16
STEPS = BATCH // PER_STEP


def _sig(x):
    return 0.5 + 0.5 * jnp.tanh(0.5 * x)


def _rtvf_body(idx_ref, *refs):
    c_ref, b_ref, v_ref, ns_ref, nh_ref = refs[3 * PER_STEP:3 * PER_STEP + 5]
    out_ref = refs[-1]
    step = pl.program_id(0)

    for k in range(PER_STEP):
        mmu_ref, mlv_ref, h_ref = refs[3 * k:3 * k + 3]
        f = idx_ref[step * PER_STEP + k]
        c = c_ref[0, f]

        th = jnp.tanh(0.5 * mmu_ref[0])
        a = 0.5 - 0.5 * th
        e = jnp.exp(mlv_ref[0])
        es = e * ns_ref[k]
        eh = e * nh_ref[k]

        for ch in range(N_CH):
            s = b_ref[ch] + c * v_ref[0, ch]
            hc = h_ref[0, ch]
            t = a * (s - hc) + hc + s * s * es + hc * hc * eh
            out_ref[k, ch] = _sig(t)


def kernel(index, img, B, V, C, Mmu, Mlv, H, noise_S, noise_H):
    del img
    idx = index.astype(jnp.int32)
    ht = jnp.transpose(H, (0, 3, 1, 2))
    vt = jnp.transpose(V, (0, 3, 1, 2))
    bt = jnp.transpose(B, (2, 0, 1))
    ct = jnp.transpose(C, (1, 0))
    ns = noise_S.reshape(BATCH, N_PIX, N_PIX)
    nh = noise_H.reshape(BATCH, N_PIX, N_PIX)

    def gspec(k):
        return lambda b, i: (i[b * PER_STEP + k], 0, 0)

    def gspec4(k):
        return lambda b, i: (i[b * PER_STEP + k], 0, 0, 0)

    in_specs = []
    operands = []
    for k in range(PER_STEP):
        in_specs.append(pl.BlockSpec((1, N_PIX, N_PIX), gspec(k)))
        in_specs.append(pl.BlockSpec((1, N_PIX, N_PIX), gspec(k)))
        in_specs.append(pl.BlockSpec((1, N_CH, N_PIX, N_PIX), gspec4(k)))
        operands.extend([Mmu, Mlv, ht])
    in_specs.extend([
        pl.BlockSpec(memory_space=pltpu.SMEM),
        pl.BlockSpec((N_CH, N_PIX, N_PIX), lambda b, i: (0, 0, 0)),
        pl.BlockSpec((1, N_CH, N_PIX, N_PIX),
                     lambda b, i: (0, 0, 0, 0)),
        pl.BlockSpec((PER_STEP, N_PIX, N_PIX), lambda b, i: (b, 0, 0)),
        pl.BlockSpec((PER_STEP, N_PIX, N_PIX), lambda b, i: (b, 0, 0)),
    ])
    operands.extend([ct, bt, vt, ns, nh])

    grid_spec = pltpu.PrefetchScalarGridSpec(
        num_scalar_prefetch=1,
        grid=(STEPS,),
        in_specs=in_specs,
        out_specs=pl.BlockSpec((PER_STEP, N_CH, N_PIX, N_PIX),
                               lambda b, i: (b, 0, 0, 0)),
    )

    out = pl.pallas_call(
        _rtvf_body,
        grid_spec=grid_spec,
        out_shape=jax.ShapeDtypeStruct((BATCH, N_CH, N_PIX, N_PIX),
                                       jnp.float32),
        compiler_params=pltpu.CompilerParams(
            dimension_semantics=("arbitrary",),
        ),
    )(idx, *operands)

    return jnp.transpose(out, (0, 2, 3, 1))
